# baseline (speedup 1.0000x reference)
import functools

import jax
import jax.numpy as jnp
from jax import lax
from jax.experimental import pallas as pl
from jax.experimental.pallas import tpu as pltpu
from jax.experimental.pallas import tpu_sc as plsc

N_USERS = 40000
N_ITEMS = 10000
N_NODES = 100000
N_EDGES = 1600000
D = 32

NC, NS, L = 2, 16, 16
HALF = N_NODES // NC
RPT = 3128
RPT_LAST = HALF - (NS - 1) * RPT
CHUNK = 128
SB = 128
NJ = SB // CHUNK

_f32 = jnp.float32
_i32 = jnp.int32


def _enc_body(v_ref, t_ref, wv1, bv1, wv2, bv2, wt1, bt1, wt2, bt2, wd, bd,
              o_ref):
    h = jnp.maximum(
        jnp.dot(v_ref[...], wv1[...], preferred_element_type=_f32) + bv1[...],
        0.0)
    ve = jnp.dot(jnp.dot(h, wv2[...], preferred_element_type=_f32) + bv2[...],
                 wd[...], preferred_element_type=_f32) + bd[...]
    ht = jnp.maximum(
        jnp.dot(t_ref[...], wt1[...], preferred_element_type=_f32) + bt1[...],
        0.0)
    te = jnp.dot(jnp.dot(ht, wt2[...], preferred_element_type=_f32) + bt2[...],
                 wd[...], preferred_element_type=_f32) + bd[...]
    o_ref[...] = (ve + te) * 0.5


def _encoder(visual, text, Wv1, bv1, Wv2, bv2, Wt1, bt1, Wt2, bt2, Wd, bd):
    BI = 1000
    grid = (N_ITEMS // BI,)

    def full(shape):
        return pl.BlockSpec(shape, lambda i: tuple(0 for _ in shape))

    return pl.pallas_call(
        _enc_body,
        grid=grid,
        in_specs=[
            pl.BlockSpec((BI, 2048), lambda i: (i, 0)),
            pl.BlockSpec((BI, 300), lambda i: (i, 0)),
            full((2048, 512)), full((1, 512)),
            full((512, D)), full((1, D)),
            full((300, 256)), full((1, 256)),
            full((256, D)), full((1, D)),
            full((D, D)), full((1, D)),
        ],
        out_specs=pl.BlockSpec((BI, D), lambda i: (i, 0)),
        out_shape=jax.ShapeDtypeStruct((N_ITEMS, D), _f32),
    )(visual, text, Wv1, bv1.reshape(1, -1), Wv2, bv2.reshape(1, -1),
      Wt1, bt1.reshape(1, -1), Wt2, bt2.reshape(1, -1), Wd, bd.reshape(1, -1))


def _comb_body(cur_ref, nb_ref, wct, wcb, bc, o_ref):
    y = (jnp.dot(cur_ref[...], wct[...], preferred_element_type=_f32)
         + jnp.dot(nb_ref[...], wcb[...], preferred_element_type=_f32)
         + bc[...])
    o_ref[...] = jnp.where(y >= 0, y, 0.01 * y)


def _combine(cur, nb, Wc, bc):
    BN = 2000
    grid = (N_NODES // BN,)

    def full(shape):
        return pl.BlockSpec(shape, lambda i: tuple(0 for _ in shape))

    wcb = Wc[D:]
    return pl.pallas_call(
        _comb_body,
        grid=grid,
        in_specs=[
            pl.BlockSpec((BN, D), lambda i: (i, 0)),
            pl.BlockSpec((BN, D), lambda i: (i, 0)),
            full((D, D)), full((D, D)), full((1, D)),
        ],
        out_specs=pl.BlockSpec((BN, D), lambda i: (i, 0)),
        out_shape=jax.ShapeDtypeStruct((N_NODES, D), _f32),
    )(cur, nb, Wc[:D], wcb, bc.reshape(1, -1))


_MESH = plsc.VectorSubcoreMesh(core_axis_name="c", subcore_axis_name="s",
                               num_cores=NC, num_subcores=NS)


def _buf(shape, dty):
    return [pltpu.VMEM(shape, dty) for _ in range(2)]


@functools.partial(
    pl.kernel,
    out_type=jax.ShapeDtypeStruct((N_NODES, D), _f32),
    mesh=_MESH,
    compiler_params=pltpu.CompilerParams(needs_layout_passes=False,
                                         use_tc_tiling_on_sc=False),
    scratch_types=[
        pltpu.VMEM((L,), _i32),
        _buf((SB,), _i32),
        _buf((SB,), _f32),
        _buf((SB,), _i32),
        _buf((NJ, CHUNK), _i32),
        _buf((SB,), _f32),
        _buf((SB, D), _f32),
        _buf((SB, D), _f32),
        pltpu.VMEM_SHARED((HALF, D), _f32),
        [pltpu.SemaphoreType.DMA for _ in range(2)],
        [pltpu.SemaphoreType.DMA for _ in range(2)],
        [pltpu.SemaphoreType.DMA for _ in range(2)],
    ],
)
def _spmm_k(table_h, col_h, val_h, row_h, split_h, z_h, out_h,
            splitv, colv, valv, rowv, rloc, vmsk, rows, rows2, acc,
            lsem, gsem, ssem):
    c = lax.axis_index("c")
    t = lax.axis_index("s")
    rbase = c * HALF

    pltpu.sync_copy(split_h, splitv)
    s = splitv[...][0]

    lo = jnp.where(c == 0, _i32(0), s)
    hi = jnp.where(c == 0, s, _i32(N_EDGES))
    cnt = hi - lo
    tlo = lo + (cnt * t) // NS
    tnext = lo + (cnt * (t + 1)) // NS
    tlo8 = (tlo // 8) * 8
    nsc = (tnext - tlo8 + SB - 1) // SB

    def sb_base(m):
        return pl.multiple_of(
            jnp.minimum(tlo8 + m * SB, _i32(N_EDGES - SB)), 8)

    def fire_lin(m, b):
        base = sb_base(m)
        pltpu.async_copy(col_h.at[pl.ds(base, SB)], colv[b], lsem[b])
        pltpu.async_copy(val_h.at[pl.ds(base, SB)], valv[b], lsem[b])
        pltpu.async_copy(row_h.at[pl.ds(base, SB)], rowv[b], lsem[b])

    def drain_lin(m, b):
        base = sb_base(m)
        pltpu.make_async_copy(col_h.at[pl.ds(base, SB)], colv[b], lsem[b]).wait()
        pltpu.make_async_copy(val_h.at[pl.ds(base, SB)], valv[b], lsem[b]).wait()
        pltpu.make_async_copy(row_h.at[pl.ds(base, SB)], rowv[b], lsem[b]).wait()

    def drain_scat(b):
        def dw(j, cc):
            pltpu.make_async_copy(rows2[b].at[pl.ds(j * CHUNK, CHUNK)],
                                  acc.at[rloc[b].at[j]], ssem[b]).wait()
            return cc
        lax.fori_loop(0, NJ, dw, 0)

    fire_lin(0, 0)

    @pl.when(nsc >= 1)
    def _():
        fire_lin(1, 1)

    @pl.when(t < NS - 1)
    def _():
        pltpu.sync_copy(z_h, acc.at[pl.ds(t * RPT, RPT)])

    @pl.when(t == NS - 1)
    def _():
        pltpu.sync_copy(z_h.at[pl.ds(0, RPT_LAST)],
                        acc.at[pl.ds(t * RPT, RPT_LAST)])

    plsc.subcore_barrier()

    def iter_i(i, b):
        o = 1 - b

        @pl.when(i <= nsc)
        def _():
            drain_lin(i, b)

            @pl.when(i >= 2)
            def _():
                drain_scat(b)

            @pl.when(i < nsc)
            def _():
                def gf(j, cc):
                    pltpu.async_copy(
                        table_h.at[colv[b].at[pl.ds(j * CHUNK, CHUNK)]],
                        rows[b].at[pl.ds(j * CHUNK, CHUNK)], gsem[b])
                    return cc
                lax.fori_loop(0, NJ, gf, 0)

            @pl.when(i >= 1)
            def _():
                base_p = sb_base(i - 1)
                lo_p = jnp.maximum(tlo, tlo8 + (i - 1) * SB)

                def gd(j, cc):
                    pltpu.make_async_copy(
                        table_h.at[colv[o].at[pl.ds(j * CHUNK, CHUNK)]],
                        rows[o].at[pl.ds(j * CHUNK, CHUNK)], gsem[o]).wait()
                    return cc
                lax.fori_loop(0, NJ, gd, 0)

                def mk(j, cc):
                    for g in range(CHUNK // L):
                        q = j * CHUNK + g * L
                        ev = lax.iota(_i32, L) + (q + base_p)
                        msk = (ev >= lo_p) & (ev < tnext)
                        v = jnp.where(msk, valv[o][pl.ds(q, L)], 0.0)
                        r = rowv[o][pl.ds(q, L)] - rbase
                        r = jnp.minimum(jnp.maximum(r, 0), HALF - 1)
                        vmsk[o][pl.ds(q, L)] = v
                        rloc[o][j, pl.ds(g * L, L)] = r
                    return cc
                lax.fori_loop(0, NJ, mk, 0)

                @pl.when(i + 1 <= nsc)
                def _():
                    fire_lin(i + 1, o)

                def pc(j, cc):
                    for g in range(CHUNK // L):
                        q = j * CHUNK + g * L
                        v = vmsk[o][pl.ds(q, L)]
                        for k in range(L):
                            e = q + k
                            vb = jnp.full((L,), v[k], _f32)
                            a = rows[o][e, pl.ds(0, L)]
                            bb = rows[o][e, pl.ds(L, L)]
                            rows2[o][e, pl.ds(0, L)] = a * vb
                            rows2[o][e, pl.ds(L, L)] = bb * vb
                    pltpu.async_copy(rows2[o].at[pl.ds(j * CHUNK, CHUNK)],
                                     acc.at[rloc[o].at[j]], ssem[o], add=True)
                    return cc
                lax.fori_loop(0, NJ, pc, 0)

    def pair(k, cc):
        iter_i(2 * k, 0)
        iter_i(2 * k + 1, 1)
        return cc

    lax.fori_loop(0, (nsc + 2) // 2, pair, 0)

    plast = (nsc - 1) % 2

    @pl.when((nsc >= 1) & (plast == 0))
    def _():
        drain_scat(0)

    @pl.when((nsc >= 1) & (plast == 1))
    def _():
        drain_scat(1)

    plsc.subcore_barrier()

    @pl.when(t < NS - 1)
    def _():
        pltpu.sync_copy(acc.at[pl.ds(t * RPT, RPT)],
                        out_h.at[pl.ds(rbase + t * RPT, RPT)])

    @pl.when(t == NS - 1)
    def _():
        pltpu.sync_copy(acc.at[pl.ds(t * RPT, RPT_LAST)],
                        out_h.at[pl.ds(rbase + t * RPT, RPT_LAST)])


def _spmm(table, colp, valp, rowp, splits, zinit):
    return _spmm_k(table, colp, valp, rowp, splits, zinit)


def kernel(visual_features, text_features, embedding, edge_val,
           Wv1, bv1, Wv2, bv2, Wt1, bt1, Wt2, bt2, Wd, bd,
           Wc0, bc0, Wc1, bc1, edge_row, edge_col):
    fused = _encoder(visual_features, text_features,
                     Wv1, bv1, Wv2, bv2, Wt1, bt1, Wt2, bt2, Wd, bd)
    ego = lax.dynamic_update_slice(embedding, fused, (N_USERS, 0))

    s = jnp.searchsorted(edge_row, _i32(HALF), side="left").astype(_i32)
    splits = jnp.full((L,), s, _i32)
    zinit = jnp.zeros((RPT, D), _f32)

    nb1 = _spmm(ego, edge_col, edge_val, edge_row, splits, zinit)
    cur1 = _combine(ego, nb1, Wc0, bc0)
    nb2 = _spmm(cur1, edge_col, edge_val, edge_row, splits, zinit)
    cur2 = _combine(cur1, nb2, Wc1, bc1)
    return jnp.concatenate([ego, cur1, cur2], axis=-1)


# speedup vs baseline: 1.1179x; 1.1179x over previous
import functools

import jax
import jax.numpy as jnp
from jax import lax
from jax.experimental import pallas as pl
from jax.experimental.pallas import tpu as pltpu
from jax.experimental.pallas import tpu_sc as plsc

N_USERS = 40000
N_ITEMS = 10000
N_NODES = 100000
N_EDGES = 1600000
D = 32

NC, NS, L = 2, 16, 16
HALF = N_NODES // NC
RPT = 3128
RPT_LAST = HALF - (NS - 1) * RPT
CHUNK = 96
SB = 192
NJ = SB // CHUNK

_f32 = jnp.float32
_i32 = jnp.int32


def _enc_body(v_ref, t_ref, wv1, bv1, wv2, bv2, wt1, bt1, wt2, bt2, wd, bd,
              o_ref):
    h = jnp.maximum(
        jnp.dot(v_ref[...], wv1[...], preferred_element_type=_f32) + bv1[...],
        0.0)
    ve = jnp.dot(jnp.dot(h, wv2[...], preferred_element_type=_f32) + bv2[...],
                 wd[...], preferred_element_type=_f32) + bd[...]
    ht = jnp.maximum(
        jnp.dot(t_ref[...], wt1[...], preferred_element_type=_f32) + bt1[...],
        0.0)
    te = jnp.dot(jnp.dot(ht, wt2[...], preferred_element_type=_f32) + bt2[...],
                 wd[...], preferred_element_type=_f32) + bd[...]
    o_ref[...] = (ve + te) * 0.5


def _encoder(visual, text, Wv1, bv1, Wv2, bv2, Wt1, bt1, Wt2, bt2, Wd, bd):
    BI = 1000
    grid = (N_ITEMS // BI,)

    def full(shape):
        return pl.BlockSpec(shape, lambda i: tuple(0 for _ in shape))

    return pl.pallas_call(
        _enc_body,
        grid=grid,
        in_specs=[
            pl.BlockSpec((BI, 2048), lambda i: (i, 0)),
            pl.BlockSpec((BI, 300), lambda i: (i, 0)),
            full((2048, 512)), full((1, 512)),
            full((512, D)), full((1, D)),
            full((300, 256)), full((1, 256)),
            full((256, D)), full((1, D)),
            full((D, D)), full((1, D)),
        ],
        out_specs=pl.BlockSpec((BI, D), lambda i: (i, 0)),
        out_shape=jax.ShapeDtypeStruct((N_ITEMS, D), _f32),
    )(visual, text, Wv1, bv1.reshape(1, -1), Wv2, bv2.reshape(1, -1),
      Wt1, bt1.reshape(1, -1), Wt2, bt2.reshape(1, -1), Wd, bd.reshape(1, -1))


def _comb_body(cur_ref, nb_ref, wct, wcb, bc, o_ref):
    y = (jnp.dot(cur_ref[...], wct[...], preferred_element_type=_f32)
         + jnp.dot(nb_ref[...], wcb[...], preferred_element_type=_f32)
         + bc[...])
    o_ref[...] = jnp.where(y >= 0, y, 0.01 * y)


def _combine(cur, nb, Wc, bc):
    BN = 2000
    grid = (N_NODES // BN,)

    def full(shape):
        return pl.BlockSpec(shape, lambda i: tuple(0 for _ in shape))

    wcb = Wc[D:]
    return pl.pallas_call(
        _comb_body,
        grid=grid,
        in_specs=[
            pl.BlockSpec((BN, D), lambda i: (i, 0)),
            pl.BlockSpec((BN, D), lambda i: (i, 0)),
            full((D, D)), full((D, D)), full((1, D)),
        ],
        out_specs=pl.BlockSpec((BN, D), lambda i: (i, 0)),
        out_shape=jax.ShapeDtypeStruct((N_NODES, D), _f32),
    )(cur, nb, Wc[:D], wcb, bc.reshape(1, -1))


_MESH = plsc.VectorSubcoreMesh(core_axis_name="c", subcore_axis_name="s",
                               num_cores=NC, num_subcores=NS)


def _buf(shape, dty):
    return [pltpu.VMEM(shape, dty) for _ in range(2)]


@functools.partial(
    pl.kernel,
    out_type=jax.ShapeDtypeStruct((N_NODES, D), _f32),
    mesh=_MESH,
    compiler_params=pltpu.CompilerParams(needs_layout_passes=False,
                                         use_tc_tiling_on_sc=False),
    scratch_types=[
        pltpu.VMEM((L,), _i32),
        _buf((SB,), _i32),
        _buf((SB,), _f32),
        _buf((SB,), _i32),
        _buf((NJ, CHUNK), _i32),
        _buf((SB,), _f32),
        _buf((SB, D), _f32),
        _buf((SB, D), _f32),
        pltpu.VMEM_SHARED((HALF, D), _f32),
        [pltpu.SemaphoreType.DMA for _ in range(2)],
        [pltpu.SemaphoreType.DMA for _ in range(2)],
        [pltpu.SemaphoreType.DMA for _ in range(2)],
    ],
)
def _spmm_k(table_h, col_h, val_h, row_h, split_h, z_h, out_h,
            splitv, colv, valv, rowv, rloc, vmsk, rows, rows2, acc,
            lsem, gsem, ssem):
    c = lax.axis_index("c")
    t = lax.axis_index("s")
    rbase = c * HALF

    pltpu.sync_copy(split_h, splitv)
    s = splitv[...][0]

    lo = jnp.where(c == 0, _i32(0), s)
    hi = jnp.where(c == 0, s, _i32(N_EDGES))
    cnt = hi - lo
    tlo = lo + (cnt * t) // NS
    tnext = lo + (cnt * (t + 1)) // NS
    tlo8 = (tlo // 8) * 8
    nsc = (tnext - tlo8 + SB - 1) // SB

    def sb_base(m):
        return pl.multiple_of(
            jnp.minimum(tlo8 + m * SB, _i32(N_EDGES - SB)), 8)

    def fire_lin(m, b):
        base = sb_base(m)
        pltpu.async_copy(col_h.at[pl.ds(base, SB)], colv[b], lsem[b])
        pltpu.async_copy(val_h.at[pl.ds(base, SB)], valv[b], lsem[b])
        pltpu.async_copy(row_h.at[pl.ds(base, SB)], rowv[b], lsem[b])

    def drain_lin(m, b):
        base = sb_base(m)
        pltpu.make_async_copy(col_h.at[pl.ds(base, SB)], colv[b], lsem[b]).wait()
        pltpu.make_async_copy(val_h.at[pl.ds(base, SB)], valv[b], lsem[b]).wait()
        pltpu.make_async_copy(row_h.at[pl.ds(base, SB)], rowv[b], lsem[b]).wait()

    def drain_scat(b):
        def dw(j, cc):
            pltpu.make_async_copy(rows2[b].at[pl.ds(j * CHUNK, CHUNK)],
                                  acc.at[rloc[b].at[j]], ssem[b]).wait()
            return cc
        lax.fori_loop(0, NJ, dw, 0)

    fire_lin(0, 0)

    @pl.when(nsc >= 1)
    def _():
        fire_lin(1, 1)

    @pl.when(t < NS - 1)
    def _():
        pltpu.sync_copy(z_h, acc.at[pl.ds(t * RPT, RPT)])

    @pl.when(t == NS - 1)
    def _():
        pltpu.sync_copy(z_h.at[pl.ds(0, RPT_LAST)],
                        acc.at[pl.ds(t * RPT, RPT_LAST)])

    plsc.subcore_barrier()

    def iter_i(i, b):
        o = 1 - b

        @pl.when(i <= nsc)
        def _():
            drain_lin(i, b)

            @pl.when(i >= 2)
            def _():
                drain_scat(b)

            @pl.when(i < nsc)
            def _():
                def gf(j, cc):
                    pltpu.async_copy(
                        table_h.at[colv[b].at[pl.ds(j * CHUNK, CHUNK)]],
                        rows[b].at[pl.ds(j * CHUNK, CHUNK)], gsem[b])
                    return cc
                lax.fori_loop(0, NJ, gf, 0)

            @pl.when(i >= 1)
            def _():
                base_p = sb_base(i - 1)
                lo_p = jnp.maximum(tlo, tlo8 + (i - 1) * SB)

                def gd(j, cc):
                    pltpu.make_async_copy(
                        table_h.at[colv[o].at[pl.ds(j * CHUNK, CHUNK)]],
                        rows[o].at[pl.ds(j * CHUNK, CHUNK)], gsem[o]).wait()
                    return cc
                lax.fori_loop(0, NJ, gd, 0)

                def mk(j, cc):
                    for g in range(CHUNK // L):
                        q = j * CHUNK + g * L
                        ev = lax.iota(_i32, L) + (q + base_p)
                        msk = (ev >= lo_p) & (ev < tnext)
                        v = jnp.where(msk, valv[o][pl.ds(q, L)], 0.0)
                        r = rowv[o][pl.ds(q, L)] - rbase
                        r = jnp.minimum(jnp.maximum(r, 0), HALF - 1)
                        vmsk[o][pl.ds(q, L)] = v
                        rloc[o][j, pl.ds(g * L, L)] = r
                    return cc
                lax.fori_loop(0, NJ, mk, 0)

                @pl.when(i + 1 <= nsc)
                def _():
                    fire_lin(i + 1, o)

                def pc(j, cc):
                    for g in range(CHUNK // L):
                        q = j * CHUNK + g * L
                        v = vmsk[o][pl.ds(q, L)]
                        for k in range(L):
                            e = q + k
                            vb = jnp.full((L,), v[k], _f32)
                            a = rows[o][e, pl.ds(0, L)]
                            bb = rows[o][e, pl.ds(L, L)]
                            rows2[o][e, pl.ds(0, L)] = a * vb
                            rows2[o][e, pl.ds(L, L)] = bb * vb
                    pltpu.async_copy(rows2[o].at[pl.ds(j * CHUNK, CHUNK)],
                                     acc.at[rloc[o].at[j]], ssem[o], add=True)
                    return cc
                lax.fori_loop(0, NJ, pc, 0)

    def pair(k, cc):
        iter_i(2 * k, 0)
        iter_i(2 * k + 1, 1)
        return cc

    lax.fori_loop(0, (nsc + 2) // 2, pair, 0)

    plast = (nsc - 1) % 2

    @pl.when((nsc >= 1) & (plast == 0))
    def _():
        drain_scat(0)

    @pl.when((nsc >= 1) & (plast == 1))
    def _():
        drain_scat(1)

    plsc.subcore_barrier()

    @pl.when(t < NS - 1)
    def _():
        pltpu.sync_copy(acc.at[pl.ds(t * RPT, RPT)],
                        out_h.at[pl.ds(rbase + t * RPT, RPT)])

    @pl.when(t == NS - 1)
    def _():
        pltpu.sync_copy(acc.at[pl.ds(t * RPT, RPT_LAST)],
                        out_h.at[pl.ds(rbase + t * RPT, RPT_LAST)])


def _spmm(table, colp, valp, rowp, splits, zinit):
    return _spmm_k(table, colp, valp, rowp, splits, zinit)


def kernel(visual_features, text_features, embedding, edge_val,
           Wv1, bv1, Wv2, bv2, Wt1, bt1, Wt2, bt2, Wd, bd,
           Wc0, bc0, Wc1, bc1, edge_row, edge_col):
    fused = _encoder(visual_features, text_features,
                     Wv1, bv1, Wv2, bv2, Wt1, bt1, Wt2, bt2, Wd, bd)
    ego = lax.dynamic_update_slice(embedding, fused, (N_USERS, 0))

    s = jnp.searchsorted(edge_row, _i32(HALF), side="left").astype(_i32)
    splits = jnp.full((L,), s, _i32)
    zinit = jnp.zeros((RPT, D), _f32)

    nb1 = _spmm(ego, edge_col, edge_val, edge_row, splits, zinit)
    cur1 = _combine(ego, nb1, Wc0, bc0)
    nb2 = _spmm(cur1, edge_col, edge_val, edge_row, splits, zinit)
    cur2 = _combine(cur1, nb2, Wc1, bc1)
    return jnp.concatenate([ego, cur1, cur2], axis=-1)
